# baseline (speedup 1.0000x reference)
import functools

import jax
import jax.numpy as jnp
from jax import lax
from jax.experimental import pallas as pl
from jax.experimental.pallas import tpu as pltpu
from jax.experimental.pallas import tpu_sc as plsc


@functools.cache
def _make_sc_gather(V, D, B):
    info = plsc.get_sparse_core_info()
    NW = info.num_cores * info.num_subcores
    assert B % NW == 0 and (B // NW) % 8 == 0
    b_per_w = B // NW
    mesh = plsc.VectorSubcoreMesh(core_axis_name="c", subcore_axis_name="s")

    @functools.partial(
        pl.kernel,
        mesh=mesh,
        out_type=jax.ShapeDtypeStruct((B, D), jnp.float32),
        scratch_types=[
            pltpu.VMEM((b_per_w,), jnp.int32),
            pltpu.VMEM((b_per_w, D), jnp.float32),
            pltpu.SemaphoreType.DMA,
        ],
        compiler_params=pltpu.CompilerParams(use_tc_tiling_on_sc=False),
    )
    def gather(table_hbm, idx_hbm, out_hbm, idx_v, rows_v, sem):
        wid = lax.axis_index("s") * info.num_cores + lax.axis_index("c")
        base = wid * b_per_w
        pltpu.sync_copy(idx_hbm.at[pl.ds(base, b_per_w)], idx_v)
        pltpu.async_copy(table_hbm.at[idx_v], rows_v, sem).wait()
        pltpu.sync_copy(rows_v, out_hbm.at[pl.ds(base, b_per_w)])

    return gather


def _matmul_body(x_ref, w_ref, o_ref):
    o_ref[...] = lax.dot_general(
        x_ref[...], w_ref[...],
        dimension_numbers=(((1,), (1,)), ((), ())),
        preferred_element_type=jnp.float32,
    )


def _projection(x, weight, v_tile=2048):
    B, D = x.shape
    V = weight.shape[0]
    return pl.pallas_call(
        _matmul_body,
        grid=(pl.cdiv(V, v_tile),),
        in_specs=[
            pl.BlockSpec((B, D), lambda j: (0, 0)),
            pl.BlockSpec((v_tile, D), lambda j: (j, 0)),
        ],
        out_specs=pl.BlockSpec((B, v_tile), lambda j: (0, j)),
        out_shape=jax.ShapeDtypeStruct((B, V), jnp.float32),
    )(x, weight)


def kernel(indices, embeddings, weight):
    B = indices.shape[0]
    V, D = embeddings.shape
    x = _make_sc_gather(V, D, B)(embeddings, indices.astype(jnp.int32))
    return _projection(x, weight)


# speedup vs baseline: 2.3512x; 2.3512x over previous
import functools

import jax
import jax.numpy as jnp
from jax import lax
from jax.experimental import pallas as pl
from jax.experimental.pallas import tpu as pltpu
from jax.experimental.pallas import tpu_sc as plsc


@functools.cache
def _make_sc_gather(V, D, B):
    info = plsc.get_sparse_core_info()
    NW = info.num_cores * info.num_subcores
    assert B % NW == 0 and (B // NW) % 8 == 0
    b_per_w = B // NW
    mesh = plsc.VectorSubcoreMesh(core_axis_name="c", subcore_axis_name="s")

    @functools.partial(
        pl.kernel,
        mesh=mesh,
        out_type=jax.ShapeDtypeStruct((B, D), jnp.float32),
        scratch_types=[
            pltpu.VMEM((b_per_w,), jnp.int32),
            pltpu.VMEM((b_per_w, D), jnp.float32),
            pltpu.SemaphoreType.DMA,
        ],
        compiler_params=pltpu.CompilerParams(use_tc_tiling_on_sc=False),
    )
    def gather(table_hbm, idx_hbm, out_hbm, idx_v, rows_v, sem):
        wid = lax.axis_index("s") * info.num_cores + lax.axis_index("c")
        base = wid * b_per_w
        pltpu.sync_copy(idx_hbm.at[pl.ds(base, b_per_w)], idx_v)
        pltpu.async_copy(table_hbm.at[idx_v], rows_v, sem).wait()
        pltpu.sync_copy(rows_v, out_hbm.at[pl.ds(base, b_per_w)])

    return gather


def _matmul_body(w_ref, x_ref, o_ref):
    o_ref[...] = lax.dot_general(
        w_ref[...].astype(jnp.bfloat16), x_ref[...].astype(jnp.bfloat16),
        dimension_numbers=(((1,), (1,)), ((), ())),
        preferred_element_type=jnp.float32,
    )


def _projection_t(x, weight, v_tile=2048):
    B, D = x.shape
    V = weight.shape[0]
    return pl.pallas_call(
        _matmul_body,
        grid=(pl.cdiv(V, v_tile),),
        in_specs=[
            pl.BlockSpec((v_tile, D), lambda j: (j, 0)),
            pl.BlockSpec((B, D), lambda j: (0, 0)),
        ],
        out_specs=pl.BlockSpec((v_tile, B), lambda j: (j, 0)),
        out_shape=jax.ShapeDtypeStruct((V, B), jnp.float32),
    )(weight, x)


def kernel(indices, embeddings, weight):
    B = indices.shape[0]
    V, D = embeddings.shape
    x = _make_sc_gather(V, D, B)(embeddings, indices.astype(jnp.int32))
    return _projection_t(x, weight).T


# speedup vs baseline: 2.8109x; 1.1955x over previous
import functools

import jax
import jax.numpy as jnp
from jax import lax
from jax.experimental import pallas as pl
from jax.experimental.pallas import tpu as pltpu
from jax.experimental.pallas import tpu_sc as plsc


@functools.cache
def _make_sc_gather(V, D, B):
    info = plsc.get_sparse_core_info()
    NW = info.num_cores * info.num_subcores
    assert B % NW == 0 and (B // NW) % 8 == 0
    b_per_w = B // NW
    mesh = plsc.VectorSubcoreMesh(core_axis_name="c", subcore_axis_name="s")

    @functools.partial(
        pl.kernel,
        mesh=mesh,
        out_type=jax.ShapeDtypeStruct((B, D), jnp.float32),
        scratch_types=[
            pltpu.VMEM((b_per_w,), jnp.int32),
            pltpu.VMEM((b_per_w, D), jnp.float32),
            pltpu.SemaphoreType.DMA,
        ],
        compiler_params=pltpu.CompilerParams(use_tc_tiling_on_sc=False),
    )
    def gather(table_hbm, idx_hbm, out_hbm, idx_v, rows_v, sem):
        wid = lax.axis_index("s") * info.num_cores + lax.axis_index("c")
        base = wid * b_per_w
        pltpu.sync_copy(idx_hbm.at[pl.ds(base, b_per_w)], idx_v)
        pltpu.async_copy(table_hbm.at[idx_v], rows_v, sem).wait()
        pltpu.sync_copy(rows_v, out_hbm.at[pl.ds(base, b_per_w)])

    return gather


def _matmul_body(wt_ref, x_ref, o_ref):
    o_ref[...] = lax.dot_general(
        wt_ref[...].astype(jnp.bfloat16), x_ref[...].astype(jnp.bfloat16),
        dimension_numbers=(((0,), (1,)), ((), ())),
        preferred_element_type=jnp.float32,
    )


def _projection_t(x, wt, v_tile=2048):
    B, D = x.shape
    V = wt.shape[1]
    return pl.pallas_call(
        _matmul_body,
        grid=(pl.cdiv(V, v_tile),),
        in_specs=[
            pl.BlockSpec((D, v_tile), lambda j: (0, j)),
            pl.BlockSpec((B, D), lambda j: (0, 0)),
        ],
        out_specs=pl.BlockSpec((v_tile, B), lambda j: (j, 0)),
        out_shape=jax.ShapeDtypeStruct((V, B), jnp.float32),
    )(wt, x)


def kernel(indices, embeddings, weight):
    B = indices.shape[0]
    V, D = embeddings.shape
    x = _make_sc_gather(V, D, B)(embeddings, indices.astype(jnp.int32))
    return _projection_t(x, weight.T).T


# speedup vs baseline: 3.6003x; 1.2808x over previous
import functools

import jax
import jax.numpy as jnp
from jax import lax
from jax.experimental import pallas as pl
from jax.experimental.pallas import tpu as pltpu
from jax.experimental.pallas import tpu_sc as plsc


@functools.cache
def _make_sc_gather_native(V, D, B, slab_w=128, nbuf=8):
    info = plsc.get_sparse_core_info()
    NW = info.num_cores * info.num_subcores
    L = info.num_lanes
    assert B % NW == 0 and (B // NW) % 8 == 0 and D % L == 0
    b_per_w = B // NW
    mesh = plsc.VectorSubcoreMesh(core_axis_name="c", subcore_axis_name="s")

    @functools.partial(
        pl.kernel,
        mesh=mesh,
        out_type=jax.ShapeDtypeStruct((B * D,), jnp.float32),
        scratch_types=[
            pltpu.VMEM((b_per_w,), jnp.int32),
            pltpu.VMEM((nbuf, D, slab_w), jnp.float32),
            pltpu.VMEM((b_per_w * D,), jnp.float32),
            pltpu.SemaphoreType.DMA((nbuf,)),
        ],
        compiler_params=pltpu.CompilerParams(
            use_tc_tiling_on_sc=True, needs_layout_passes=False
        ),
    )
    def gather(tableT_hbm, idx_hbm, out_hbm, idx_v, slabs, rows, sems):
        wid = lax.axis_index("s") * info.num_cores + lax.axis_index("c")
        base = wid * b_per_w
        pltpu.sync_copy(idx_hbm.at[pl.ds(base, b_per_w)], idx_v)

        lane_iota = lax.broadcasted_iota(jnp.int32, (L,), 0)

        def idx_scalar(k):
            vec = idx_v[pl.ds((k // L) * L, L)]
            return jnp.sum(jnp.where(lane_iota == (k % L), vec, 0))

        scalars = [idx_scalar(k) for k in range(b_per_w)]

        def issue(k):
            start = pl.multiple_of((scalars[k] // slab_w) * slab_w, slab_w)
            return pltpu.async_copy(
                tableT_hbm.at[:, pl.ds(start, slab_w)],
                slabs.at[k % nbuf],
                sems.at[k % nbuf],
            )

        copies = {}
        for k in range(min(nbuf, b_per_w)):
            copies[k] = issue(k)
        for k in range(b_per_w):
            copies[k].wait()
            lane = jnp.broadcast_to(scalars[k] % slab_w, (L,))
            for g in range(D // L):
                row_ids = lane_iota + g * L
                vals = plsc.load_gather(slabs.at[k % nbuf], [row_ids, lane])
                rows[pl.ds(k * D + g * L, L)] = vals
            if k + nbuf < b_per_w:
                copies[k + nbuf] = issue(k + nbuf)
        pltpu.sync_copy(rows, out_hbm.at[pl.ds(base * D, b_per_w * D)])

    return gather


@functools.cache
def _make_sc_gather(V, D, B):
    info = plsc.get_sparse_core_info()
    NW = info.num_cores * info.num_subcores
    assert B % NW == 0 and (B // NW) % 8 == 0
    b_per_w = B // NW
    mesh = plsc.VectorSubcoreMesh(core_axis_name="c", subcore_axis_name="s")

    @functools.partial(
        pl.kernel,
        mesh=mesh,
        out_type=jax.ShapeDtypeStruct((B, D), jnp.float32),
        scratch_types=[
            pltpu.VMEM((b_per_w,), jnp.int32),
            pltpu.VMEM((b_per_w, D), jnp.float32),
            pltpu.SemaphoreType.DMA,
        ],
        compiler_params=pltpu.CompilerParams(use_tc_tiling_on_sc=False),
    )
    def gather(table_hbm, idx_hbm, out_hbm, idx_v, rows_v, sem):
        wid = lax.axis_index("s") * info.num_cores + lax.axis_index("c")
        base = wid * b_per_w
        pltpu.sync_copy(idx_hbm.at[pl.ds(base, b_per_w)], idx_v)
        pltpu.async_copy(table_hbm.at[idx_v], rows_v, sem).wait()
        pltpu.sync_copy(rows_v, out_hbm.at[pl.ds(base, b_per_w)])

    return gather


def _matmul_body(wt_ref, x_ref, o_ref):
    o_ref[...] = lax.dot_general(
        wt_ref[...].astype(jnp.bfloat16), x_ref[...].astype(jnp.bfloat16),
        dimension_numbers=(((0,), (1,)), ((), ())),
        preferred_element_type=jnp.float32,
    )


def _projection_t(x, wt, v_tile=2048):
    B, D = x.shape
    V = wt.shape[1]
    return pl.pallas_call(
        _matmul_body,
        grid=(pl.cdiv(V, v_tile),),
        in_specs=[
            pl.BlockSpec((D, v_tile), lambda j: (0, j)),
            pl.BlockSpec((B, D), lambda j: (0, 0)),
        ],
        out_specs=pl.BlockSpec((v_tile, B), lambda j: (j, 0)),
        out_shape=jax.ShapeDtypeStruct((V, B), jnp.float32),
    )(wt, x)


def kernel(indices, embeddings, weight):
    B = indices.shape[0]
    V, D = embeddings.shape
    xf = _make_sc_gather_native(V, D, B)(embeddings.T, indices.astype(jnp.int32))
    return _projection_t(xf.reshape(B, D), weight.T).T


# speedup vs baseline: 3.6499x; 1.0138x over previous
import functools

import jax
import jax.numpy as jnp
from jax import lax
from jax.experimental import pallas as pl
from jax.experimental.pallas import tpu as pltpu
from jax.experimental.pallas import tpu_sc as plsc


@functools.cache
def _make_sc_gather_native(V, D, B, slab_w=128, nbuf=8):
    info = plsc.get_sparse_core_info()
    NW = info.num_cores * info.num_subcores
    L = info.num_lanes
    assert B % NW == 0 and (B // NW) % 8 == 0 and D % L == 0
    b_per_w = B // NW
    mesh = plsc.VectorSubcoreMesh(core_axis_name="c", subcore_axis_name="s")

    @functools.partial(
        pl.kernel,
        mesh=mesh,
        out_type=jax.ShapeDtypeStruct((B * D,), jnp.float32),
        scratch_types=[
            pltpu.VMEM((b_per_w,), jnp.int32),
            pltpu.VMEM((nbuf, D, slab_w), jnp.float32),
            pltpu.VMEM((b_per_w * D,), jnp.float32),
            pltpu.SemaphoreType.DMA((nbuf,)),
        ],
        compiler_params=pltpu.CompilerParams(
            use_tc_tiling_on_sc=True, needs_layout_passes=False
        ),
    )
    def gather(tableT_hbm, idx_hbm, out_hbm, idx_v, slabs, rows, sems):
        wid = lax.axis_index("s") * info.num_cores + lax.axis_index("c")
        base = wid * b_per_w
        pltpu.sync_copy(idx_hbm.at[pl.ds(base, b_per_w)], idx_v)

        lane_iota = lax.broadcasted_iota(jnp.int32, (L,), 0)

        def idx_scalar(k):
            vec = idx_v[pl.ds((k // L) * L, L)]
            return jnp.sum(jnp.where(lane_iota == (k % L), vec, 0))

        scalars = [idx_scalar(k) for k in range(b_per_w)]

        def issue(k):
            start = pl.multiple_of((scalars[k] // slab_w) * slab_w, slab_w)
            return pltpu.async_copy(
                tableT_hbm.at[:, pl.ds(start, slab_w)],
                slabs.at[k % nbuf],
                sems.at[k % nbuf],
            )

        copies = {}
        for k in range(min(nbuf, b_per_w)):
            copies[k] = issue(k)
        for k in range(b_per_w):
            copies[k].wait()
            lane = jnp.broadcast_to(scalars[k] % slab_w, (L,))
            for g in range(D // L):
                row_ids = lane_iota + g * L
                vals = plsc.load_gather(slabs.at[k % nbuf], [row_ids, lane])
                rows[pl.ds(k * D + g * L, L)] = vals
            if k + nbuf < b_per_w:
                copies[k + nbuf] = issue(k + nbuf)
        pltpu.sync_copy(rows, out_hbm.at[pl.ds(base * D, b_per_w * D)])

    return gather


@functools.cache
def _make_sc_gather(V, D, B):
    info = plsc.get_sparse_core_info()
    NW = info.num_cores * info.num_subcores
    assert B % NW == 0 and (B // NW) % 8 == 0
    b_per_w = B // NW
    mesh = plsc.VectorSubcoreMesh(core_axis_name="c", subcore_axis_name="s")

    @functools.partial(
        pl.kernel,
        mesh=mesh,
        out_type=jax.ShapeDtypeStruct((B, D), jnp.float32),
        scratch_types=[
            pltpu.VMEM((b_per_w,), jnp.int32),
            pltpu.VMEM((b_per_w, D), jnp.float32),
            pltpu.SemaphoreType.DMA,
        ],
        compiler_params=pltpu.CompilerParams(use_tc_tiling_on_sc=False),
    )
    def gather(table_hbm, idx_hbm, out_hbm, idx_v, rows_v, sem):
        wid = lax.axis_index("s") * info.num_cores + lax.axis_index("c")
        base = wid * b_per_w
        pltpu.sync_copy(idx_hbm.at[pl.ds(base, b_per_w)], idx_v)
        pltpu.async_copy(table_hbm.at[idx_v], rows_v, sem).wait()
        pltpu.sync_copy(rows_v, out_hbm.at[pl.ds(base, b_per_w)])

    return gather


def _matmul_body(wt_ref, x_ref, o_ref):
    o_ref[...] = lax.dot_general(
        wt_ref[...].astype(jnp.bfloat16), x_ref[...].astype(jnp.bfloat16),
        dimension_numbers=(((0,), (1,)), ((), ())),
        preferred_element_type=jnp.float32,
    )


def _projection_t(x, wt, v_tile=4096):
    B, D = x.shape
    V = wt.shape[1]
    return pl.pallas_call(
        _matmul_body,
        grid=(pl.cdiv(V, v_tile),),
        in_specs=[
            pl.BlockSpec((D, v_tile), lambda j: (0, j)),
            pl.BlockSpec((B, D), lambda j: (0, 0)),
        ],
        out_specs=pl.BlockSpec((v_tile, B), lambda j: (j, 0)),
        out_shape=jax.ShapeDtypeStruct((V, B), jnp.float32),
    )(wt, x)


def kernel(indices, embeddings, weight):
    B = indices.shape[0]
    V, D = embeddings.shape
    xf = _make_sc_gather_native(V, D, B)(embeddings.T, indices.astype(jnp.int32))
    return _projection_t(xf.reshape(B, D), weight.T).T


# speedup vs baseline: 3.6682x; 1.0050x over previous
import functools

import jax
import jax.numpy as jnp
from jax import lax
from jax.experimental import pallas as pl
from jax.experimental.pallas import tpu as pltpu
from jax.experimental.pallas import tpu_sc as plsc


@functools.cache
def _make_sc_gather_native(V, D, B, slab_w=128, nbuf=8):
    info = plsc.get_sparse_core_info()
    NW = info.num_cores * info.num_subcores
    L = info.num_lanes
    assert B % NW == 0 and (B // NW) % 8 == 0 and D % L == 0
    b_per_w = B // NW
    mesh = plsc.VectorSubcoreMesh(core_axis_name="c", subcore_axis_name="s")

    @functools.partial(
        pl.kernel,
        mesh=mesh,
        out_type=jax.ShapeDtypeStruct((B * D,), jnp.float32),
        scratch_types=[
            pltpu.VMEM((b_per_w,), jnp.int32),
            pltpu.VMEM((nbuf, D, slab_w), jnp.float32),
            pltpu.VMEM((b_per_w * D,), jnp.float32),
            pltpu.SemaphoreType.DMA((nbuf,)),
        ],
        compiler_params=pltpu.CompilerParams(
            use_tc_tiling_on_sc=True, needs_layout_passes=False
        ),
    )
    def gather(tableT_hbm, idx_hbm, out_hbm, idx_v, slabs, rows, sems):
        wid = lax.axis_index("s") * info.num_cores + lax.axis_index("c")
        base = wid * b_per_w
        pltpu.sync_copy(idx_hbm.at[pl.ds(base, b_per_w)], idx_v)

        lane_iota = lax.broadcasted_iota(jnp.int32, (L,), 0)

        def idx_scalar(k):
            vec = idx_v[pl.ds((k // L) * L, L)]
            return jnp.sum(jnp.where(lane_iota == (k % L), vec, 0))

        scalars = [idx_scalar(k) for k in range(b_per_w)]

        def issue(k):
            start = pl.multiple_of((scalars[k] // slab_w) * slab_w, slab_w)
            return pltpu.async_copy(
                tableT_hbm.at[:, pl.ds(start, slab_w)],
                slabs.at[k % nbuf],
                sems.at[k % nbuf],
            )

        copies = {}
        for k in range(min(nbuf, b_per_w)):
            copies[k] = issue(k)
        for k in range(b_per_w):
            copies[k].wait()
            lane = jnp.broadcast_to(scalars[k] % slab_w, (L,))
            for g in range(D // L):
                row_ids = lane_iota + g * L
                vals = plsc.load_gather(slabs.at[k % nbuf], [row_ids, lane])
                rows[pl.ds(k * D + g * L, L)] = vals
            if k + nbuf < b_per_w:
                copies[k + nbuf] = issue(k + nbuf)
        pltpu.sync_copy(rows, out_hbm.at[pl.ds(base * D, b_per_w * D)])

    return gather


@functools.cache
def _make_sc_gather(V, D, B):
    info = plsc.get_sparse_core_info()
    NW = info.num_cores * info.num_subcores
    assert B % NW == 0 and (B // NW) % 8 == 0
    b_per_w = B // NW
    mesh = plsc.VectorSubcoreMesh(core_axis_name="c", subcore_axis_name="s")

    @functools.partial(
        pl.kernel,
        mesh=mesh,
        out_type=jax.ShapeDtypeStruct((B, D), jnp.float32),
        scratch_types=[
            pltpu.VMEM((b_per_w,), jnp.int32),
            pltpu.VMEM((b_per_w, D), jnp.float32),
            pltpu.SemaphoreType.DMA,
        ],
        compiler_params=pltpu.CompilerParams(use_tc_tiling_on_sc=False),
    )
    def gather(table_hbm, idx_hbm, out_hbm, idx_v, rows_v, sem):
        wid = lax.axis_index("s") * info.num_cores + lax.axis_index("c")
        base = wid * b_per_w
        pltpu.sync_copy(idx_hbm.at[pl.ds(base, b_per_w)], idx_v)
        pltpu.async_copy(table_hbm.at[idx_v], rows_v, sem).wait()
        pltpu.sync_copy(rows_v, out_hbm.at[pl.ds(base, b_per_w)])

    return gather


def _matmul_body(wt_ref, x_ref, o_ref):
    o_ref[...] = lax.dot_general(
        wt_ref[...].astype(jnp.bfloat16), x_ref[...].astype(jnp.bfloat16),
        dimension_numbers=(((0,), (1,)), ((), ())),
        preferred_element_type=jnp.float32,
    )


def _projection_t(x, wt, v_tile=5120):
    B, D = x.shape
    V = wt.shape[1]
    return pl.pallas_call(
        _matmul_body,
        grid=(pl.cdiv(V, v_tile),),
        in_specs=[
            pl.BlockSpec((D, v_tile), lambda j: (0, j)),
            pl.BlockSpec((B, D), lambda j: (0, 0)),
        ],
        out_specs=pl.BlockSpec((v_tile, B), lambda j: (j, 0)),
        out_shape=jax.ShapeDtypeStruct((V, B), jnp.float32),
    )(wt, x)


def kernel(indices, embeddings, weight):
    B = indices.shape[0]
    V, D = embeddings.shape
    xf = _make_sc_gather_native(V, D, B)(embeddings.T, indices.astype(jnp.int32))
    return _projection_t(xf.reshape(B, D), weight.T).T


# speedup vs baseline: 3.6904x; 1.0061x over previous
import functools

import jax
import jax.numpy as jnp
from jax import lax
from jax.experimental import pallas as pl
from jax.experimental.pallas import tpu as pltpu
from jax.experimental.pallas import tpu_sc as plsc


@functools.cache
def _make_sc_gather_native(V, D, B, slab_w=128, nbuf=8):
    info = plsc.get_sparse_core_info()
    NW = info.num_cores * info.num_subcores
    L = info.num_lanes
    assert B % NW == 0 and (B // NW) % 8 == 0 and D % L == 0
    b_per_w = B // NW
    mesh = plsc.VectorSubcoreMesh(core_axis_name="c", subcore_axis_name="s")

    @functools.partial(
        pl.kernel,
        mesh=mesh,
        out_type=jax.ShapeDtypeStruct((B, D), jnp.float32),
        scratch_types=[
            pltpu.VMEM((b_per_w,), jnp.int32),
            pltpu.VMEM((nbuf, D, slab_w), jnp.float32),
            pltpu.VMEM((b_per_w, D), jnp.float32),
            pltpu.SemaphoreType.DMA((nbuf,)),
        ],
        compiler_params=pltpu.CompilerParams(
            use_tc_tiling_on_sc=True, needs_layout_passes=False
        ),
    )
    def gather(tableT_hbm, idx_hbm, out_hbm, idx_v, slabs, rows, sems):
        wid = lax.axis_index("s") * info.num_cores + lax.axis_index("c")
        base = wid * b_per_w
        pltpu.sync_copy(idx_hbm.at[pl.ds(base, b_per_w)], idx_v)

        lane_iota = lax.broadcasted_iota(jnp.int32, (L,), 0)

        def idx_scalar(k):
            vec = idx_v[pl.ds((k // L) * L, L)]
            return jnp.sum(jnp.where(lane_iota == (k % L), vec, 0))

        scalars = [idx_scalar(k) for k in range(b_per_w)]

        def issue(k):
            start = pl.multiple_of((scalars[k] // slab_w) * slab_w, slab_w)
            return pltpu.async_copy(
                tableT_hbm.at[:, pl.ds(start, slab_w)],
                slabs.at[k % nbuf],
                sems.at[k % nbuf],
            )

        copies = {}
        for k in range(min(nbuf, b_per_w)):
            copies[k] = issue(k)
        for k in range(b_per_w):
            copies[k].wait()
            lane = jnp.broadcast_to(scalars[k] % slab_w, (L,))
            for g in range(D // L):
                row_ids = lane_iota + g * L
                vals = plsc.load_gather(slabs.at[k % nbuf], [row_ids, lane])
                rows[k, pl.ds(g * L, L)] = vals
            if k + nbuf < b_per_w:
                copies[k + nbuf] = issue(k + nbuf)
        pltpu.sync_copy(rows, out_hbm.at[pl.ds(base, b_per_w)])

    return gather


@functools.cache
def _make_sc_gather(V, D, B):
    info = plsc.get_sparse_core_info()
    NW = info.num_cores * info.num_subcores
    assert B % NW == 0 and (B // NW) % 8 == 0
    b_per_w = B // NW
    mesh = plsc.VectorSubcoreMesh(core_axis_name="c", subcore_axis_name="s")

    @functools.partial(
        pl.kernel,
        mesh=mesh,
        out_type=jax.ShapeDtypeStruct((B, D), jnp.float32),
        scratch_types=[
            pltpu.VMEM((b_per_w,), jnp.int32),
            pltpu.VMEM((b_per_w, D), jnp.float32),
            pltpu.SemaphoreType.DMA,
        ],
        compiler_params=pltpu.CompilerParams(use_tc_tiling_on_sc=False),
    )
    def gather(table_hbm, idx_hbm, out_hbm, idx_v, rows_v, sem):
        wid = lax.axis_index("s") * info.num_cores + lax.axis_index("c")
        base = wid * b_per_w
        pltpu.sync_copy(idx_hbm.at[pl.ds(base, b_per_w)], idx_v)
        pltpu.async_copy(table_hbm.at[idx_v], rows_v, sem).wait()
        pltpu.sync_copy(rows_v, out_hbm.at[pl.ds(base, b_per_w)])

    return gather


def _matmul_body(wt_ref, x_ref, o_ref):
    o_ref[...] = lax.dot_general(
        wt_ref[...].astype(jnp.bfloat16), x_ref[...].astype(jnp.bfloat16),
        dimension_numbers=(((0,), (1,)), ((), ())),
        preferred_element_type=jnp.float32,
    )


def _projection_t(x, wt, v_tile=5120):
    B, D = x.shape
    V = wt.shape[1]
    return pl.pallas_call(
        _matmul_body,
        grid=(pl.cdiv(V, v_tile),),
        in_specs=[
            pl.BlockSpec((D, v_tile), lambda j: (0, j)),
            pl.BlockSpec((B, D), lambda j: (0, 0)),
        ],
        out_specs=pl.BlockSpec((v_tile, B), lambda j: (j, 0)),
        out_shape=jax.ShapeDtypeStruct((V, B), jnp.float32),
    )(wt, x)


def kernel(indices, embeddings, weight):
    B = indices.shape[0]
    V, D = embeddings.shape
    x = _make_sc_gather_native(V, D, B)(embeddings.T, indices.astype(jnp.int32))
    return _projection_t(x, weight.T).T
